# baseline (device time: 62705 ns/iter reference)
import jax
import jax.numpy as jnp
from jax import lax
from jax.experimental import pallas as pl
from jax.experimental.pallas import tpu as pltpu

N_DEV = 4
B, SQ, SKV = 2, 512, 512
HQ, DH = 8, 64
WIN = 128
D_MODEL = 768
D_HEADS = HQ * DH
ROWS = B * SQ
CHUNK = ROWS // (2 * N_DEV)


def kernel(x, Wq, K_ext, V_ext, Wo):
    def body(x_ref, wq_ref, k_ref, v_ref, wo_ref, out_ref,
             rsr_ref, rsl_ref,
             rs_send_r, rs_recv_r, rs_send_l, rs_recv_l,
             ag_send_r, ag_recv_r, ag_send_l, ag_recv_l):
        my = lax.axis_index("i")
        left = lax.rem(my + N_DEV - 1, N_DEV)
        right = lax.rem(my + 1, N_DEV)

        barrier_sem = pltpu.get_barrier_semaphore()
        for nbr in (left, right):
            pl.semaphore_signal(barrier_sem, inc=1, device_id=(nbr,),
                                device_id_type=pl.DeviceIdType.MESH)
        pl.semaphore_wait(barrier_sem, 2)

        x_flat = x_ref[...].reshape(ROWS, D_MODEL)
        wq_my = wq_ref[:, pl.ds(my * D_HEADS, D_HEADS)]
        q = jnp.dot(x_flat, wq_my, preferred_element_type=jnp.float32)

        qi = lax.broadcasted_iota(jnp.int32, (SQ, SKV), 0)
        ki = lax.broadcasted_iota(jnp.int32, (SQ, SKV), 1)
        mask = jnp.abs(qi - ki) <= WIN

        ctx_rows = []
        for b in range(B):
            head_cols = []
            for h in range(HQ):
                qbh = q[b * SQ:(b + 1) * SQ, h * DH:(h + 1) * DH]
                kbh = k_ref[b, :, h, :]
                vbh = v_ref[b, :, h, :]
                s = jnp.dot(qbh, kbh.T,
                            preferred_element_type=jnp.float32) * 0.125
                s = jnp.where(mask, s, -1e9)
                m = jnp.max(s, axis=-1, keepdims=True)
                w = jnp.exp(s - m)
                w = w / jnp.sum(w, axis=-1, keepdims=True)
                head_cols.append(
                    jnp.dot(w, vbh, preferred_element_type=jnp.float32))
            ctx_rows.append(jnp.concatenate(head_cols, axis=-1))
        ctx = jnp.concatenate(ctx_rows, axis=0)

        wo_my = wo_ref[pl.ds(my * D_HEADS, D_HEADS), :]
        out_ref[...] = jnp.dot(ctx, wo_my, preferred_element_type=jnp.float32)

        def r_off(c):
            return lax.rem(c + 2 * N_DEV, N_DEV) * CHUNK

        def l_off(c):
            return N_DEV * CHUNK + lax.rem(c + 2 * N_DEV, N_DEV) * CHUNK

        for t in range(N_DEV - 1):
            rd_r = pltpu.make_async_remote_copy(
                src_ref=out_ref.at[pl.ds(r_off(my - t), CHUNK), :],
                dst_ref=rsr_ref.at[t],
                send_sem=rs_send_r.at[t], recv_sem=rs_recv_r.at[t],
                device_id=(right,), device_id_type=pl.DeviceIdType.MESH,
            )
            rd_l = pltpu.make_async_remote_copy(
                src_ref=out_ref.at[pl.ds(l_off(my + t), CHUNK), :],
                dst_ref=rsl_ref.at[t],
                send_sem=rs_send_l.at[t], recv_sem=rs_recv_l.at[t],
                device_id=(left,), device_id_type=pl.DeviceIdType.MESH,
            )
            rd_r.start()
            rd_l.start()
            rd_r.wait()
            rd_l.wait()
            out_ref[pl.ds(r_off(my - t - 1), CHUNK), :] += rsr_ref[t]
            out_ref[pl.ds(l_off(my + t + 1), CHUNK), :] += rsl_ref[t]

        for t in range(N_DEV - 1):
            so_r = r_off(my + 1 - t)
            so_l = l_off(my - 1 + t)
            ag_r = pltpu.make_async_remote_copy(
                src_ref=out_ref.at[pl.ds(so_r, CHUNK), :],
                dst_ref=out_ref.at[pl.ds(so_r, CHUNK), :],
                send_sem=ag_send_r.at[t], recv_sem=ag_recv_r.at[t],
                device_id=(right,), device_id_type=pl.DeviceIdType.MESH,
            )
            ag_l = pltpu.make_async_remote_copy(
                src_ref=out_ref.at[pl.ds(so_l, CHUNK), :],
                dst_ref=out_ref.at[pl.ds(so_l, CHUNK), :],
                send_sem=ag_send_l.at[t], recv_sem=ag_recv_l.at[t],
                device_id=(left,), device_id_type=pl.DeviceIdType.MESH,
            )
            ag_r.start()
            ag_l.start()
            ag_r.wait()
            ag_l.wait()

    out_flat = pl.pallas_call(
        body,
        out_shape=jax.ShapeDtypeStruct((ROWS, D_MODEL), jnp.float32),
        in_specs=[pl.BlockSpec(memory_space=pltpu.VMEM)] * 5,
        out_specs=pl.BlockSpec(memory_space=pltpu.VMEM),
        scratch_shapes=[
            pltpu.VMEM((N_DEV - 1, CHUNK, D_MODEL), jnp.float32),
            pltpu.VMEM((N_DEV - 1, CHUNK, D_MODEL), jnp.float32),
            pltpu.SemaphoreType.DMA((N_DEV - 1,)),
            pltpu.SemaphoreType.DMA((N_DEV - 1,)),
            pltpu.SemaphoreType.DMA((N_DEV - 1,)),
            pltpu.SemaphoreType.DMA((N_DEV - 1,)),
            pltpu.SemaphoreType.DMA((N_DEV - 1,)),
            pltpu.SemaphoreType.DMA((N_DEV - 1,)),
            pltpu.SemaphoreType.DMA((N_DEV - 1,)),
            pltpu.SemaphoreType.DMA((N_DEV - 1,)),
        ],
        compiler_params=pltpu.CompilerParams(collective_id=0),
    )(x, Wq, K_ext, V_ext, Wo)
    return out_flat.reshape(B, SQ, D_MODEL)


# device time: 52858 ns/iter; 1.1863x vs baseline; 1.1863x over previous
import jax
import jax.numpy as jnp
from jax import lax
from jax.experimental import pallas as pl
from jax.experimental.pallas import tpu as pltpu

N_DEV = 4
B, SQ, SKV = 2, 512, 512
HQ, DH = 8, 64
WIN = 128
D_MODEL = 768
D_HEADS = HQ * DH
ROWS = B * SQ
CHUNK = ROWS // (2 * N_DEV)

BF = jnp.bfloat16
F32 = jnp.float32


def _dot(a, b):
    return jnp.dot(a.astype(BF), b.astype(BF), preferred_element_type=F32)


def kernel(x, Wq, K_ext, V_ext, Wo):
    def body(x_ref, wq_ref, k_ref, v_ref, wo_ref, out_ref,
             rsr_ref, rsl_ref, rs_stage_r, rs_stage_l,
             agr_ref, agl_ref, ag_stage_r, ag_stage_l,
             rs_send_r, rs_recv_r, rs_send_l, rs_recv_l,
             ag_send_r, ag_recv_r, ag_send_l, ag_recv_l):
        my = lax.axis_index("i")
        left = lax.rem(my + N_DEV - 1, N_DEV)
        right = lax.rem(my + 1, N_DEV)

        barrier_sem = pltpu.get_barrier_semaphore()
        for nbr in (left, right):
            pl.semaphore_signal(barrier_sem, inc=1, device_id=(nbr,),
                                device_id_type=pl.DeviceIdType.MESH)
        pl.semaphore_wait(barrier_sem, 2)

        x_flat = x_ref[...].reshape(ROWS, D_MODEL)
        wq_my = wq_ref[:, pl.ds(my * D_HEADS, D_HEADS)]
        q = _dot(x_flat, wq_my)

        qi = lax.broadcasted_iota(jnp.int32, (SQ, SKV), 0)
        ki = lax.broadcasted_iota(jnp.int32, (SQ, SKV), 1)
        mask = jnp.abs(qi - ki) <= WIN

        ctx_rows = []
        for b in range(B):
            head_cols = []
            for h in range(HQ):
                qbh = q[b * SQ:(b + 1) * SQ, h * DH:(h + 1) * DH]
                kbh = k_ref[b, :, h, :]
                vbh = v_ref[b, :, h, :]
                s = _dot(qbh, kbh.T) * 0.125
                s = jnp.where(mask, s, -1e9)
                m = jnp.max(s, axis=-1, keepdims=True)
                w = jnp.exp(s - m)
                w = w / jnp.sum(w, axis=-1, keepdims=True)
                head_cols.append(_dot(w, vbh))
            ctx_rows.append(jnp.concatenate(head_cols, axis=-1))
        ctx = jnp.concatenate(ctx_rows, axis=0)

        wo_my = wo_ref[pl.ds(my * D_HEADS, D_HEADS), :]
        out_ref[...] = _dot(ctx, wo_my)

        def r_off(c):
            return lax.rem(c + 2 * N_DEV, N_DEV) * CHUNK

        def l_off(c):
            return N_DEV * CHUNK + lax.rem(c + 2 * N_DEV, N_DEV) * CHUNK

        for t in range(N_DEV - 1):
            rs_stage_r[...] = out_ref[pl.ds(r_off(my - t), CHUNK), :].astype(BF)
            rs_stage_l[...] = out_ref[pl.ds(l_off(my + t), CHUNK), :].astype(BF)
            rd_r = pltpu.make_async_remote_copy(
                src_ref=rs_stage_r,
                dst_ref=rsr_ref.at[t],
                send_sem=rs_send_r.at[t], recv_sem=rs_recv_r.at[t],
                device_id=(right,), device_id_type=pl.DeviceIdType.MESH,
            )
            rd_l = pltpu.make_async_remote_copy(
                src_ref=rs_stage_l,
                dst_ref=rsl_ref.at[t],
                send_sem=rs_send_l.at[t], recv_sem=rs_recv_l.at[t],
                device_id=(left,), device_id_type=pl.DeviceIdType.MESH,
            )
            rd_r.start()
            rd_l.start()
            rd_r.wait()
            rd_l.wait()
            out_ref[pl.ds(r_off(my - t - 1), CHUNK), :] += rsr_ref[t].astype(F32)
            out_ref[pl.ds(l_off(my + t + 1), CHUNK), :] += rsl_ref[t].astype(F32)

        ag_stage_r[...] = out_ref[pl.ds(r_off(my + 1), CHUNK), :].astype(BF)
        ag_stage_l[...] = out_ref[pl.ds(l_off(my - 1), CHUNK), :].astype(BF)
        for t in range(N_DEV - 1):
            src_r = ag_stage_r if t == 0 else agr_ref.at[t - 1]
            src_l = ag_stage_l if t == 0 else agl_ref.at[t - 1]
            ag_r = pltpu.make_async_remote_copy(
                src_ref=src_r,
                dst_ref=agr_ref.at[t],
                send_sem=ag_send_r.at[t], recv_sem=ag_recv_r.at[t],
                device_id=(right,), device_id_type=pl.DeviceIdType.MESH,
            )
            ag_l = pltpu.make_async_remote_copy(
                src_ref=src_l,
                dst_ref=agl_ref.at[t],
                send_sem=ag_send_l.at[t], recv_sem=ag_recv_l.at[t],
                device_id=(left,), device_id_type=pl.DeviceIdType.MESH,
            )
            ag_r.start()
            ag_l.start()
            ag_r.wait()
            ag_l.wait()
            out_ref[pl.ds(r_off(my - t), CHUNK), :] = agr_ref[t].astype(F32)
            out_ref[pl.ds(l_off(my + t), CHUNK), :] = agl_ref[t].astype(F32)

    out_flat = pl.pallas_call(
        body,
        out_shape=jax.ShapeDtypeStruct((ROWS, D_MODEL), F32),
        in_specs=[pl.BlockSpec(memory_space=pltpu.VMEM)] * 5,
        out_specs=pl.BlockSpec(memory_space=pltpu.VMEM),
        scratch_shapes=[
            pltpu.VMEM((N_DEV - 1, CHUNK, D_MODEL), BF),
            pltpu.VMEM((N_DEV - 1, CHUNK, D_MODEL), BF),
            pltpu.VMEM((CHUNK, D_MODEL), BF),
            pltpu.VMEM((CHUNK, D_MODEL), BF),
            pltpu.VMEM((N_DEV - 1, CHUNK, D_MODEL), BF),
            pltpu.VMEM((N_DEV - 1, CHUNK, D_MODEL), BF),
            pltpu.VMEM((CHUNK, D_MODEL), BF),
            pltpu.VMEM((CHUNK, D_MODEL), BF),
            pltpu.SemaphoreType.DMA((N_DEV - 1,)),
            pltpu.SemaphoreType.DMA((N_DEV - 1,)),
            pltpu.SemaphoreType.DMA((N_DEV - 1,)),
            pltpu.SemaphoreType.DMA((N_DEV - 1,)),
            pltpu.SemaphoreType.DMA((N_DEV - 1,)),
            pltpu.SemaphoreType.DMA((N_DEV - 1,)),
            pltpu.SemaphoreType.DMA((N_DEV - 1,)),
            pltpu.SemaphoreType.DMA((N_DEV - 1,)),
        ],
        compiler_params=pltpu.CompilerParams(collective_id=0),
    )(x, Wq, K_ext, V_ext, Wo)
    return out_flat.reshape(B, SQ, D_MODEL)
